# Initial kernel scaffold; baseline (speedup 1.0000x reference)
import functools

import jax
import jax.numpy as jnp
from jax import lax
from jax.experimental import pallas as pl
from jax.experimental.pallas import tpu as pltpu
from jax.experimental.pallas import tpu_sc as plsc

N = 10000
E = 320000
NC = 2
NS = 16
NW = NC * NS
EPT = E // NW
CH = 100
NCH = EPT // CH
RPT = N // NS


@functools.lru_cache(maxsize=None)
def _make_spmm(D):
  mesh = plsc.VectorSubcoreMesh(core_axis_name="c", subcore_axis_name="s")

  @functools.partial(
      pl.kernel,
      out_type=jax.ShapeDtypeStruct((NC, N, D), jnp.float32),
      mesh=mesh,
      scratch_types=[
          pltpu.VMEM((NCH, CH), jnp.int32),
          pltpu.VMEM((NCH, CH), jnp.int32),
          pltpu.VMEM((CH, D), jnp.float32),
          pltpu.VMEM_SHARED((N, D), jnp.float32),
          pltpu.SemaphoreType.DMA,
      ],
  )
  def spmm(support, src3, dst3, zeros, out, src_v, dst_v, rows, acc, sem):
    cid = lax.axis_index("c")
    sid = lax.axis_index("s")
    wid = sid * NC + cid
    pltpu.sync_copy(src3.at[wid], src_v)
    pltpu.sync_copy(dst3.at[wid], dst_v)
    pltpu.sync_copy(zeros.at[pl.ds(sid * RPT, RPT)],
                    acc.at[pl.ds(sid * RPT, RPT)])
    plsc.subcore_barrier()

    def body(j, carry):
      pltpu.async_copy(support.at[src_v.at[j]], rows, sem).wait()
      pltpu.sync_copy(rows, acc.at[dst_v.at[j]], add=True)
      return carry

    lax.fori_loop(0, NCH, body, 0)
    plsc.subcore_barrier()
    pltpu.sync_copy(acc.at[pl.ds(sid * RPT, RPT)],
                    out.at[cid, pl.ds(sid * RPT, RPT)])

  return spmm


BN = 1000


def _mm_first_body(x_ref, w_ref, s_ref, b_ref, sup_ref, slf_ref):
  x = x_ref[...]
  sup_ref[...] = jnp.dot(x, w_ref[...], preferred_element_type=jnp.float32)
  slf_ref[...] = (
      jnp.dot(x, s_ref[...], preferred_element_type=jnp.float32) + b_ref[...])


def _mm_mid_body(agg_ref, slfp_ref, w_ref, s_ref, b_ref, sup_ref, slf_ref):
  x = agg_ref[0] + agg_ref[1] + slfp_ref[...]
  sup_ref[...] = jnp.dot(x, w_ref[...], preferred_element_type=jnp.float32)
  slf_ref[...] = (
      jnp.dot(x, s_ref[...], preferred_element_type=jnp.float32) + b_ref[...])


def _final_body(agg_ref, slfp_ref, o_ref):
  x = agg_ref[0] + agg_ref[1] + slfp_ref[...]
  m = jnp.max(x, axis=1, keepdims=True)
  e = jnp.exp(x - m)
  lse = jnp.log(jnp.sum(e, axis=1, keepdims=True)) + m
  o_ref[...] = x - lse


@functools.lru_cache(maxsize=None)
def _make_mm_first(DI, DO):
  return pl.pallas_call(
      _mm_first_body,
      grid=(N // BN,),
      in_specs=[
          pl.BlockSpec((BN, DI), lambda i: (i, 0)),
          pl.BlockSpec((DI, DO), lambda i: (0, 0)),
          pl.BlockSpec((DI, DO), lambda i: (0, 0)),
          pl.BlockSpec((1, DO), lambda i: (0, 0)),
      ],
      out_specs=[
          pl.BlockSpec((BN, DO), lambda i: (i, 0)),
          pl.BlockSpec((BN, DO), lambda i: (i, 0)),
      ],
      out_shape=[
          jax.ShapeDtypeStruct((N, DO), jnp.float32),
          jax.ShapeDtypeStruct((N, DO), jnp.float32),
      ],
  )


@functools.lru_cache(maxsize=None)
def _make_mm_mid(DI, DO):
  return pl.pallas_call(
      _mm_mid_body,
      grid=(N // BN,),
      in_specs=[
          pl.BlockSpec((NC, BN, DI), lambda i: (0, i, 0)),
          pl.BlockSpec((BN, DI), lambda i: (i, 0)),
          pl.BlockSpec((DI, DO), lambda i: (0, 0)),
          pl.BlockSpec((DI, DO), lambda i: (0, 0)),
          pl.BlockSpec((1, DO), lambda i: (0, 0)),
      ],
      out_specs=[
          pl.BlockSpec((BN, DO), lambda i: (i, 0)),
          pl.BlockSpec((BN, DO), lambda i: (i, 0)),
      ],
      out_shape=[
          jax.ShapeDtypeStruct((N, DO), jnp.float32),
          jax.ShapeDtypeStruct((N, DO), jnp.float32),
      ],
  )


@functools.lru_cache(maxsize=None)
def _make_final(D):
  return pl.pallas_call(
      _final_body,
      grid=(N // BN,),
      in_specs=[
          pl.BlockSpec((NC, BN, D), lambda i: (0, i, 0)),
          pl.BlockSpec((BN, D), lambda i: (i, 0)),
      ],
      out_specs=pl.BlockSpec((BN, D), lambda i: (i, 0)),
      out_shape=jax.ShapeDtypeStruct((N, D), jnp.float32),
  )


def kernel(fea, adj, W0, S0, b0, W1, S1, b1, W2, S2, b2, W3, S3, b3):
  src3 = adj[0].reshape(NW, NCH, CH)
  dst3 = adj[1].reshape(NW, NCH, CH)
  z128 = jnp.zeros((N, 128), jnp.float32)
  z64 = jnp.zeros((N, 64), jnp.float32)

  sup, slf = _make_mm_first(128, 128)(fea, W0, S0, b0.reshape(1, -1))
  agg = _make_spmm(128)(sup, src3, dst3, z128)
  sup, slf = _make_mm_mid(128, 128)(agg, slf, W1, S1, b1.reshape(1, -1))
  agg = _make_spmm(128)(sup, src3, dst3, z128)
  sup, slf = _make_mm_mid(128, 128)(agg, slf, W2, S2, b2.reshape(1, -1))
  agg = _make_spmm(128)(sup, src3, dst3, z128)
  sup, slf = _make_mm_mid(128, 64)(agg, slf, W3, S3, b3.reshape(1, -1))
  agg = _make_spmm(64)(sup, src3, dst3, z64)
  return _make_final(64)(agg, slf)


# speedup vs baseline: 6.7398x; 6.7398x over previous
import functools

import jax
import jax.numpy as jnp
from jax import lax
from jax.experimental import pallas as pl
from jax.experimental.pallas import tpu as pltpu
from jax.experimental.pallas import tpu_sc as plsc

N = 10000
E = 320000
NC = 2
NS = 16
NW = NC * NS
EPT = E // NW
CH = 100
NCH = EPT // CH
NPAD = 10240
RPT = NPAD // NS


@functools.lru_cache(maxsize=None)
def _make_spmm(D):
  mesh = plsc.VectorSubcoreMesh(core_axis_name="c", subcore_axis_name="s")

  @functools.partial(
      pl.kernel,
      out_type=jax.ShapeDtypeStruct((NC, NPAD, D), jnp.float32),
      mesh=mesh,
      scratch_types=[
          pltpu.VMEM((NCH, CH), jnp.int32),
          pltpu.VMEM((NCH, CH), jnp.int32),
          pltpu.VMEM((CH, D), jnp.float32),
          pltpu.VMEM_SHARED((NPAD, D), jnp.float32),
          pltpu.SemaphoreType.DMA,
      ],
  )
  def spmm(support, src3, dst3, zeros, out, src_v, dst_v, rows, acc, sem):
    cid = lax.axis_index("c")
    sid = lax.axis_index("s")
    wid = sid * NC + cid
    pltpu.sync_copy(src3.at[wid], src_v)
    pltpu.sync_copy(dst3.at[wid], dst_v)
    pltpu.sync_copy(zeros.at[pl.ds(sid * RPT, RPT)],
                    acc.at[pl.ds(sid * RPT, RPT)])
    plsc.subcore_barrier()

    def body(j, carry):
      pltpu.async_copy(support.at[src_v.at[j]], rows, sem).wait()
      pltpu.sync_copy(rows, acc.at[dst_v.at[j]], add=True)
      return carry

    lax.fori_loop(0, NCH, body, 0)
    plsc.subcore_barrier()
    pltpu.sync_copy(acc.at[pl.ds(sid * RPT, RPT)],
                    out.at[cid, pl.ds(sid * RPT, RPT)])

  return spmm


BN = 1000


def _mm_first_body(x_ref, w_ref, s_ref, b_ref, sup_ref, slf_ref):
  x = x_ref[...]
  sup_ref[...] = jnp.dot(x, w_ref[...], preferred_element_type=jnp.float32)
  slf_ref[...] = (
      jnp.dot(x, s_ref[...], preferred_element_type=jnp.float32) + b_ref[...])


def _mm_mid_body(agg_ref, slfp_ref, w_ref, s_ref, b_ref, sup_ref, slf_ref):
  x = agg_ref[0] + agg_ref[1] + slfp_ref[...]
  sup_ref[...] = jnp.dot(x, w_ref[...], preferred_element_type=jnp.float32)
  slf_ref[...] = (
      jnp.dot(x, s_ref[...], preferred_element_type=jnp.float32) + b_ref[...])


def _final_body(agg_ref, slfp_ref, o_ref):
  x = (agg_ref[0] + agg_ref[1] + slfp_ref[...])[:, :64]
  m = jnp.max(x, axis=1, keepdims=True)
  e = jnp.exp(x - m)
  lse = jnp.log(jnp.sum(e, axis=1, keepdims=True)) + m
  o_ref[...] = x - lse


@functools.lru_cache(maxsize=None)
def _make_mm_first(DI, DO):
  return pl.pallas_call(
      _mm_first_body,
      grid=(N // BN,),
      in_specs=[
          pl.BlockSpec((BN, DI), lambda i: (i, 0)),
          pl.BlockSpec((DI, DO), lambda i: (0, 0)),
          pl.BlockSpec((DI, DO), lambda i: (0, 0)),
          pl.BlockSpec((1, DO), lambda i: (0, 0)),
      ],
      out_specs=[
          pl.BlockSpec((BN, DO), lambda i: (i, 0)),
          pl.BlockSpec((BN, DO), lambda i: (i, 0)),
      ],
      out_shape=[
          jax.ShapeDtypeStruct((N, DO), jnp.float32),
          jax.ShapeDtypeStruct((N, DO), jnp.float32),
      ],
  )


@functools.lru_cache(maxsize=None)
def _make_mm_mid(DI, DO):
  return pl.pallas_call(
      _mm_mid_body,
      grid=(N // BN,),
      in_specs=[
          pl.BlockSpec((NC, BN, DI), lambda i: (0, i, 0)),
          pl.BlockSpec((BN, DI), lambda i: (i, 0)),
          pl.BlockSpec((DI, DO), lambda i: (0, 0)),
          pl.BlockSpec((DI, DO), lambda i: (0, 0)),
          pl.BlockSpec((1, DO), lambda i: (0, 0)),
      ],
      out_specs=[
          pl.BlockSpec((BN, DO), lambda i: (i, 0)),
          pl.BlockSpec((BN, DO), lambda i: (i, 0)),
      ],
      out_shape=[
          jax.ShapeDtypeStruct((N, DO), jnp.float32),
          jax.ShapeDtypeStruct((N, DO), jnp.float32),
      ],
  )


@functools.lru_cache(maxsize=None)
def _make_final():
  return pl.pallas_call(
      _final_body,
      grid=(N // BN,),
      in_specs=[
          pl.BlockSpec((NC, BN, 128), lambda i: (0, i, 0)),
          pl.BlockSpec((BN, 128), lambda i: (i, 0)),
      ],
      out_specs=pl.BlockSpec((BN, 64), lambda i: (i, 0)),
      out_shape=jax.ShapeDtypeStruct((N, 64), jnp.float32),
  )


def kernel(fea, adj, W0, S0, b0, W1, S1, b1, W2, S2, b2, W3, S3, b3):
  src3 = adj[0].reshape(NW, NCH, CH)
  dst3 = adj[1].reshape(NW, NCH, CH)
  z128 = jnp.zeros((NPAD, 128), jnp.float32)
  pad = jnp.zeros((128, 64), jnp.float32)
  W3p = jnp.concatenate([W3, pad], axis=1)
  S3p = jnp.concatenate([S3, pad], axis=1)
  b3p = jnp.concatenate([b3, jnp.zeros((64,), jnp.float32)])

  sup, slf = _make_mm_first(128, 128)(fea, W0, S0, b0.reshape(1, -1))
  agg = _make_spmm(128)(sup, src3, dst3, z128)
  sup, slf = _make_mm_mid(128, 128)(agg, slf, W1, S1, b1.reshape(1, -1))
  agg = _make_spmm(128)(sup, src3, dst3, z128)
  sup, slf = _make_mm_mid(128, 128)(agg, slf, W2, S2, b2.reshape(1, -1))
  agg = _make_spmm(128)(sup, src3, dst3, z128)
  sup, slf = _make_mm_mid(128, 128)(agg, slf, W3p, S3p, b3p.reshape(1, -1))
  agg = _make_spmm(128)(sup, src3, dst3, z128)
  return _make_final()(agg, slf)


# speedup vs baseline: 10.2799x; 1.5253x over previous
import functools

import jax
import jax.numpy as jnp
from jax import lax
from jax.experimental import pallas as pl
from jax.experimental.pallas import tpu as pltpu
from jax.experimental.pallas import tpu_sc as plsc

N = 10000
E = 320000
NC = 2
NS = 16
NW = NC * NS
EPT = E // NW
CH = 100
NCH = EPT // CH
NH = 2
NCH2 = NCH // NH
NPAD = 10240
RPT = NPAD // NS


@functools.lru_cache(maxsize=None)
def _make_spmm(D):
  mesh = plsc.VectorSubcoreMesh(core_axis_name="c", subcore_axis_name="s")

  @functools.partial(
      pl.kernel,
      out_type=jax.ShapeDtypeStruct((NC, NPAD, D), jnp.float32),
      mesh=mesh,
      scratch_types=[
          pltpu.VMEM((NCH2, CH), jnp.int32),
          pltpu.VMEM((NCH2, CH), jnp.int32),
          pltpu.VMEM((CH, D), jnp.float32),
          pltpu.VMEM((CH, D), jnp.float32),
          pltpu.VMEM_SHARED((NPAD, D), jnp.float32),
          pltpu.SemaphoreType.DMA,
          pltpu.SemaphoreType.DMA,
      ],
  )
  def spmm(support, src4, dst4, zeros, out,
           src_v, dst_v, rows0, rows1, acc, sem0, sem1):
    cid = lax.axis_index("c")
    sid = lax.axis_index("s")
    wid = sid * NC + cid
    pltpu.sync_copy(zeros.at[pl.ds(sid * RPT, RPT)],
                    acc.at[pl.ds(sid * RPT, RPT)])
    plsc.subcore_barrier()

    for h in range(NH):
      pltpu.sync_copy(src4.at[wid, h], src_v)
      pltpu.sync_copy(dst4.at[wid, h], dst_v)
      pltpu.async_copy(support.at[src_v.at[0]], rows0, sem0)

      def body(i, carry):
        j0 = 2 * i
        pltpu.async_copy(support.at[src_v.at[j0 + 1]], rows1, sem1)
        pltpu.make_async_copy(support.at[src_v.at[j0]], rows0, sem0).wait()
        pltpu.sync_copy(rows0, acc.at[dst_v.at[j0]], add=True)
        jn = lax.rem(j0 + 2, NCH2)
        pltpu.async_copy(support.at[src_v.at[jn]], rows0, sem0)
        pltpu.make_async_copy(support.at[src_v.at[j0 + 1]], rows1,
                              sem1).wait()
        pltpu.sync_copy(rows1, acc.at[dst_v.at[j0 + 1]], add=True)
        return carry

      lax.fori_loop(0, NCH2 // 2, body, 0)
      pltpu.make_async_copy(support.at[src_v.at[0]], rows0, sem0).wait()
    plsc.subcore_barrier()
    pltpu.sync_copy(acc.at[pl.ds(sid * RPT, RPT)],
                    out.at[cid, pl.ds(sid * RPT, RPT)])

  return spmm


BN = 1000


def _mm_first_body(x_ref, w_ref, s_ref, b_ref, sup_ref, slf_ref):
  x = x_ref[...]
  sup_ref[...] = jnp.dot(x, w_ref[...], preferred_element_type=jnp.float32)
  slf_ref[...] = (
      jnp.dot(x, s_ref[...], preferred_element_type=jnp.float32) + b_ref[...])


def _mm_mid_body(agg_ref, slfp_ref, w_ref, s_ref, b_ref, sup_ref, slf_ref):
  x = agg_ref[0] + agg_ref[1] + slfp_ref[...]
  sup_ref[...] = jnp.dot(x, w_ref[...], preferred_element_type=jnp.float32)
  slf_ref[...] = (
      jnp.dot(x, s_ref[...], preferred_element_type=jnp.float32) + b_ref[...])


def _final_body(agg_ref, slfp_ref, o_ref):
  x = (agg_ref[0] + agg_ref[1] + slfp_ref[...])[:, :64]
  m = jnp.max(x, axis=1, keepdims=True)
  e = jnp.exp(x - m)
  lse = jnp.log(jnp.sum(e, axis=1, keepdims=True)) + m
  o_ref[...] = x - lse


@functools.lru_cache(maxsize=None)
def _make_mm_first(DI, DO):
  return pl.pallas_call(
      _mm_first_body,
      grid=(N // BN,),
      in_specs=[
          pl.BlockSpec((BN, DI), lambda i: (i, 0)),
          pl.BlockSpec((DI, DO), lambda i: (0, 0)),
          pl.BlockSpec((DI, DO), lambda i: (0, 0)),
          pl.BlockSpec((1, DO), lambda i: (0, 0)),
      ],
      out_specs=[
          pl.BlockSpec((BN, DO), lambda i: (i, 0)),
          pl.BlockSpec((BN, DO), lambda i: (i, 0)),
      ],
      out_shape=[
          jax.ShapeDtypeStruct((N, DO), jnp.float32),
          jax.ShapeDtypeStruct((N, DO), jnp.float32),
      ],
  )


@functools.lru_cache(maxsize=None)
def _make_mm_mid(DI, DO):
  return pl.pallas_call(
      _mm_mid_body,
      grid=(N // BN,),
      in_specs=[
          pl.BlockSpec((NC, BN, DI), lambda i: (0, i, 0)),
          pl.BlockSpec((BN, DI), lambda i: (i, 0)),
          pl.BlockSpec((DI, DO), lambda i: (0, 0)),
          pl.BlockSpec((DI, DO), lambda i: (0, 0)),
          pl.BlockSpec((1, DO), lambda i: (0, 0)),
      ],
      out_specs=[
          pl.BlockSpec((BN, DO), lambda i: (i, 0)),
          pl.BlockSpec((BN, DO), lambda i: (i, 0)),
      ],
      out_shape=[
          jax.ShapeDtypeStruct((N, DO), jnp.float32),
          jax.ShapeDtypeStruct((N, DO), jnp.float32),
      ],
  )


@functools.lru_cache(maxsize=None)
def _make_final():
  return pl.pallas_call(
      _final_body,
      grid=(N // BN,),
      in_specs=[
          pl.BlockSpec((NC, BN, 128), lambda i: (0, i, 0)),
          pl.BlockSpec((BN, 128), lambda i: (i, 0)),
      ],
      out_specs=pl.BlockSpec((BN, 64), lambda i: (i, 0)),
      out_shape=jax.ShapeDtypeStruct((N, 64), jnp.float32),
  )


def kernel(fea, adj, W0, S0, b0, W1, S1, b1, W2, S2, b2, W3, S3, b3):
  src3 = adj[0].reshape(NW, NH, NCH2, CH)
  dst3 = adj[1].reshape(NW, NH, NCH2, CH)
  z128 = jnp.zeros((NPAD, 128), jnp.float32)
  pad = jnp.zeros((128, 64), jnp.float32)
  W3p = jnp.concatenate([W3, pad], axis=1)
  S3p = jnp.concatenate([S3, pad], axis=1)
  b3p = jnp.concatenate([b3, jnp.zeros((64,), jnp.float32)])

  sup, slf = _make_mm_first(128, 128)(fea, W0, S0, b0.reshape(1, -1))
  agg = _make_spmm(128)(sup, src3, dst3, z128)
  sup, slf = _make_mm_mid(128, 128)(agg, slf, W1, S1, b1.reshape(1, -1))
  agg = _make_spmm(128)(sup, src3, dst3, z128)
  sup, slf = _make_mm_mid(128, 128)(agg, slf, W2, S2, b2.reshape(1, -1))
  agg = _make_spmm(128)(sup, src3, dst3, z128)
  sup, slf = _make_mm_mid(128, 128)(agg, slf, W3p, S3p, b3p.reshape(1, -1))
  agg = _make_spmm(128)(sup, src3, dst3, z128)
  return _make_final()(agg, slf)


# speedup vs baseline: 10.5802x; 1.0292x over previous
import functools

import jax
import jax.numpy as jnp
from jax import lax
from jax.experimental import pallas as pl
from jax.experimental.pallas import tpu as pltpu
from jax.experimental.pallas import tpu_sc as plsc

N = 10000
E = 320000
NC = 2
NS = 16
NW = NC * NS
EPT = E // NW
CH = 125
NCH = EPT // CH
NH = 2
NCH2 = NCH // NH
NPAD = 10112
RPT = NPAD // NS


@functools.lru_cache(maxsize=None)
def _make_spmm(D):
  mesh = plsc.VectorSubcoreMesh(core_axis_name="c", subcore_axis_name="s")

  @functools.partial(
      pl.kernel,
      out_type=jax.ShapeDtypeStruct((NC, NPAD, D), jnp.float32),
      mesh=mesh,
      scratch_types=[
          pltpu.VMEM((NCH2, CH), jnp.int32),
          pltpu.VMEM((NCH2, CH), jnp.int32),
          pltpu.VMEM((CH, D), jnp.float32),
          pltpu.VMEM((CH, D), jnp.float32),
          pltpu.VMEM_SHARED((NPAD, D), jnp.float32),
          pltpu.SemaphoreType.DMA,
          pltpu.SemaphoreType.DMA,
      ],
  )
  def spmm(support, src4, dst4, zeros, out,
           src_v, dst_v, rows0, rows1, acc, sem0, sem1):
    cid = lax.axis_index("c")
    sid = lax.axis_index("s")
    wid = sid * NC + cid
    pltpu.sync_copy(zeros.at[pl.ds(sid * RPT, RPT)],
                    acc.at[pl.ds(sid * RPT, RPT)])
    plsc.subcore_barrier()

    for h in range(NH):
      pltpu.sync_copy(src4.at[wid, h], src_v)
      pltpu.sync_copy(dst4.at[wid, h], dst_v)
      pltpu.async_copy(support.at[src_v.at[0]], rows0, sem0)

      def body(i, carry):
        j0 = 2 * i
        pltpu.async_copy(support.at[src_v.at[j0 + 1]], rows1, sem1)
        pltpu.make_async_copy(support.at[src_v.at[j0]], rows0, sem0).wait()
        pltpu.sync_copy(rows0, acc.at[dst_v.at[j0]], add=True)
        jn = lax.rem(j0 + 2, NCH2)
        pltpu.async_copy(support.at[src_v.at[jn]], rows0, sem0)
        pltpu.make_async_copy(support.at[src_v.at[j0 + 1]], rows1,
                              sem1).wait()
        pltpu.sync_copy(rows1, acc.at[dst_v.at[j0 + 1]], add=True)
        return carry

      lax.fori_loop(0, NCH2 // 2, body, 0)
      pltpu.make_async_copy(support.at[src_v.at[0]], rows0, sem0).wait()
    plsc.subcore_barrier()
    pltpu.sync_copy(acc.at[pl.ds(sid * RPT, RPT)],
                    out.at[cid, pl.ds(sid * RPT, RPT)])

  return spmm


BN = 1000


def _mm_first_body(x_ref, w_ref, s_ref, b_ref, sup_ref, slf_ref):
  x = x_ref[...]
  sup_ref[...] = jnp.dot(x, w_ref[...], preferred_element_type=jnp.float32)
  slf_ref[...] = (
      jnp.dot(x, s_ref[...], preferred_element_type=jnp.float32) + b_ref[...])


def _mm_mid_body(agg_ref, slfp_ref, w_ref, s_ref, b_ref, sup_ref, slf_ref):
  x = agg_ref[0] + agg_ref[1] + slfp_ref[...]
  sup_ref[...] = jnp.dot(x, w_ref[...], preferred_element_type=jnp.float32)
  slf_ref[...] = (
      jnp.dot(x, s_ref[...], preferred_element_type=jnp.float32) + b_ref[...])


def _final_body(agg_ref, slfp_ref, o_ref):
  x = (agg_ref[0] + agg_ref[1] + slfp_ref[...])[:, :64]
  m = jnp.max(x, axis=1, keepdims=True)
  e = jnp.exp(x - m)
  lse = jnp.log(jnp.sum(e, axis=1, keepdims=True)) + m
  o_ref[...] = x - lse


@functools.lru_cache(maxsize=None)
def _make_mm_first(DI, DO):
  return pl.pallas_call(
      _mm_first_body,
      grid=(N // BN,),
      in_specs=[
          pl.BlockSpec((BN, DI), lambda i: (i, 0)),
          pl.BlockSpec((DI, DO), lambda i: (0, 0)),
          pl.BlockSpec((DI, DO), lambda i: (0, 0)),
          pl.BlockSpec((1, DO), lambda i: (0, 0)),
      ],
      out_specs=[
          pl.BlockSpec((BN, DO), lambda i: (i, 0)),
          pl.BlockSpec((BN, DO), lambda i: (i, 0)),
      ],
      out_shape=[
          jax.ShapeDtypeStruct((N, DO), jnp.float32),
          jax.ShapeDtypeStruct((N, DO), jnp.float32),
      ],
  )


@functools.lru_cache(maxsize=None)
def _make_mm_mid(DI, DO):
  return pl.pallas_call(
      _mm_mid_body,
      grid=(N // BN,),
      in_specs=[
          pl.BlockSpec((NC, BN, DI), lambda i: (0, i, 0)),
          pl.BlockSpec((BN, DI), lambda i: (i, 0)),
          pl.BlockSpec((DI, DO), lambda i: (0, 0)),
          pl.BlockSpec((DI, DO), lambda i: (0, 0)),
          pl.BlockSpec((1, DO), lambda i: (0, 0)),
      ],
      out_specs=[
          pl.BlockSpec((BN, DO), lambda i: (i, 0)),
          pl.BlockSpec((BN, DO), lambda i: (i, 0)),
      ],
      out_shape=[
          jax.ShapeDtypeStruct((N, DO), jnp.float32),
          jax.ShapeDtypeStruct((N, DO), jnp.float32),
      ],
  )


@functools.lru_cache(maxsize=None)
def _make_final():
  return pl.pallas_call(
      _final_body,
      grid=(N // BN,),
      in_specs=[
          pl.BlockSpec((NC, BN, 128), lambda i: (0, i, 0)),
          pl.BlockSpec((BN, 128), lambda i: (i, 0)),
      ],
      out_specs=pl.BlockSpec((BN, 64), lambda i: (i, 0)),
      out_shape=jax.ShapeDtypeStruct((N, 64), jnp.float32),
  )


def kernel(fea, adj, W0, S0, b0, W1, S1, b1, W2, S2, b2, W3, S3, b3):
  src3 = adj[0].reshape(NW, NH, NCH2, CH)
  dst3 = adj[1].reshape(NW, NH, NCH2, CH)
  z128 = jnp.zeros((NPAD, 128), jnp.float32)
  pad = jnp.zeros((128, 64), jnp.float32)
  W3p = jnp.concatenate([W3, pad], axis=1)
  S3p = jnp.concatenate([S3, pad], axis=1)
  b3p = jnp.concatenate([b3, jnp.zeros((64,), jnp.float32)])

  sup, slf = _make_mm_first(128, 128)(fea, W0, S0, b0.reshape(1, -1))
  agg = _make_spmm(128)(sup, src3, dst3, z128)
  sup, slf = _make_mm_mid(128, 128)(agg, slf, W1, S1, b1.reshape(1, -1))
  agg = _make_spmm(128)(sup, src3, dst3, z128)
  sup, slf = _make_mm_mid(128, 128)(agg, slf, W2, S2, b2.reshape(1, -1))
  agg = _make_spmm(128)(sup, src3, dst3, z128)
  sup, slf = _make_mm_mid(128, 128)(agg, slf, W3p, S3p, b3p.reshape(1, -1))
  agg = _make_spmm(128)(sup, src3, dst3, z128)
  return _make_final()(agg, slf)


# speedup vs baseline: 10.7519x; 1.0162x over previous
import functools

import jax
import jax.numpy as jnp
from jax import lax
from jax.experimental import pallas as pl
from jax.experimental.pallas import tpu as pltpu
from jax.experimental.pallas import tpu_sc as plsc

N = 10000
E = 320000
NC = 2
NS = 16
NW = NC * NS
EPT = E // NW
CH = 80
NCH = EPT // CH
NH = 5
NCH2 = NCH // NH
NPAD = 10112
RPT = NPAD // NS


@functools.lru_cache(maxsize=None)
def _make_spmm(D):
  mesh = plsc.VectorSubcoreMesh(core_axis_name="c", subcore_axis_name="s")

  @functools.partial(
      pl.kernel,
      out_type=jax.ShapeDtypeStruct((NC, NPAD, D), jnp.float32),
      mesh=mesh,
      scratch_types=[
          pltpu.VMEM((2, NCH2, CH), jnp.int32),
          pltpu.VMEM((2, NCH2, CH), jnp.int32),
          pltpu.VMEM((3, CH, D), jnp.float32),
          pltpu.VMEM_SHARED((NPAD, D), jnp.float32),
          [pltpu.SemaphoreType.DMA] * 3,
          [pltpu.SemaphoreType.DMA] * 3,
          pltpu.SemaphoreType.DMA,
      ],
  )
  def spmm(support, src5, dst5, zeros, out,
           src_v, dst_v, rows, acc, gsem, ssem, isem):
    cid = lax.axis_index("c")
    sid = lax.axis_index("s")
    wid = sid * NC + cid

    def g_issue(p, j, b):
      pltpu.async_copy(support.at[src_v.at[p, j]], rows.at[b], gsem[b])

    def g_wait(b):
      pltpu.make_async_copy(support.at[src_v.at[0, 0]], rows.at[b],
                            gsem[b]).wait()

    def s_issue(p, j, b):
      pltpu.async_copy(rows.at[b], acc.at[dst_v.at[p, j]], ssem[b], add=True)

    def s_wait(b):
      pltpu.make_async_copy(rows.at[b], acc.at[dst_v.at[0, 0]],
                            ssem[b]).wait()

    pltpu.sync_copy(src5.at[wid, 0], src_v.at[0])
    pltpu.sync_copy(dst5.at[wid, 0], dst_v.at[0])
    g_issue(0, 0, 0)
    g_issue(0, 1, 1)
    pltpu.sync_copy(zeros.at[pl.ds(sid * RPT, RPT)],
                    acc.at[pl.ds(sid * RPT, RPT)])
    plsc.subcore_barrier()

    for h in range(NH):
      p = h % 2
      if h + 1 < NH:
        pltpu.async_copy(src5.at[wid, h + 1], src_v.at[1 - p], isem)
        pltpu.async_copy(dst5.at[wid, h + 1], dst_v.at[1 - p], isem)
      g_wait(0)
      s_issue(p, 0, 0)
      g_issue(p, 2, 2)

      def body(i, carry):
        for k in range(3):
          j = 1 + 3 * i + k
          b = (1 + k) % 3
          bn = (b + 2) % 3
          g_wait(b)
          s_issue(p, j, b)
          s_wait(bn)
          g_issue(p, lax.rem(j + 2, NCH2), bn)
        return carry

      lax.fori_loop(0, (NCH2 - 1) // 3, body, 0)
      g_wait(1)
      g_wait(2)
      s_wait(0)
      if h + 1 < NH:
        pltpu.make_async_copy(src5.at[wid, 0], src_v.at[0], isem).wait()
        pltpu.make_async_copy(dst5.at[wid, 0], dst_v.at[0], isem).wait()
        g_issue(1 - p, 0, 0)
        g_issue(1 - p, 1, 1)
    plsc.subcore_barrier()
    pltpu.sync_copy(acc.at[pl.ds(sid * RPT, RPT)],
                    out.at[cid, pl.ds(sid * RPT, RPT)])

  return spmm


BN = 1000


def _mm_first_body(x_ref, w_ref, s_ref, b_ref, sup_ref, slf_ref):
  x = x_ref[...]
  sup_ref[...] = jnp.dot(x, w_ref[...], preferred_element_type=jnp.float32)
  slf_ref[...] = (
      jnp.dot(x, s_ref[...], preferred_element_type=jnp.float32) + b_ref[...])


def _mm_mid_body(agg_ref, slfp_ref, w_ref, s_ref, b_ref, sup_ref, slf_ref):
  x = agg_ref[0] + agg_ref[1] + slfp_ref[...]
  sup_ref[...] = jnp.dot(x, w_ref[...], preferred_element_type=jnp.float32)
  slf_ref[...] = (
      jnp.dot(x, s_ref[...], preferred_element_type=jnp.float32) + b_ref[...])


def _final_body(agg_ref, slfp_ref, o_ref):
  x = (agg_ref[0] + agg_ref[1] + slfp_ref[...])[:, :64]
  m = jnp.max(x, axis=1, keepdims=True)
  e = jnp.exp(x - m)
  lse = jnp.log(jnp.sum(e, axis=1, keepdims=True)) + m
  o_ref[...] = x - lse


@functools.lru_cache(maxsize=None)
def _make_mm_first(DI, DO):
  return pl.pallas_call(
      _mm_first_body,
      grid=(N // BN,),
      in_specs=[
          pl.BlockSpec((BN, DI), lambda i: (i, 0)),
          pl.BlockSpec((DI, DO), lambda i: (0, 0)),
          pl.BlockSpec((DI, DO), lambda i: (0, 0)),
          pl.BlockSpec((1, DO), lambda i: (0, 0)),
      ],
      out_specs=[
          pl.BlockSpec((BN, DO), lambda i: (i, 0)),
          pl.BlockSpec((BN, DO), lambda i: (i, 0)),
      ],
      out_shape=[
          jax.ShapeDtypeStruct((N, DO), jnp.float32),
          jax.ShapeDtypeStruct((N, DO), jnp.float32),
      ],
  )


@functools.lru_cache(maxsize=None)
def _make_mm_mid(DI, DO):
  return pl.pallas_call(
      _mm_mid_body,
      grid=(N // BN,),
      in_specs=[
          pl.BlockSpec((NC, BN, DI), lambda i: (0, i, 0)),
          pl.BlockSpec((BN, DI), lambda i: (i, 0)),
          pl.BlockSpec((DI, DO), lambda i: (0, 0)),
          pl.BlockSpec((DI, DO), lambda i: (0, 0)),
          pl.BlockSpec((1, DO), lambda i: (0, 0)),
      ],
      out_specs=[
          pl.BlockSpec((BN, DO), lambda i: (i, 0)),
          pl.BlockSpec((BN, DO), lambda i: (i, 0)),
      ],
      out_shape=[
          jax.ShapeDtypeStruct((N, DO), jnp.float32),
          jax.ShapeDtypeStruct((N, DO), jnp.float32),
      ],
  )


@functools.lru_cache(maxsize=None)
def _make_final():
  return pl.pallas_call(
      _final_body,
      grid=(N // BN,),
      in_specs=[
          pl.BlockSpec((NC, BN, 128), lambda i: (0, i, 0)),
          pl.BlockSpec((BN, 128), lambda i: (i, 0)),
      ],
      out_specs=pl.BlockSpec((BN, 64), lambda i: (i, 0)),
      out_shape=jax.ShapeDtypeStruct((N, 64), jnp.float32),
  )


def kernel(fea, adj, W0, S0, b0, W1, S1, b1, W2, S2, b2, W3, S3, b3):
  src3 = adj[0].reshape(NW, NH, NCH2, CH)
  dst3 = adj[1].reshape(NW, NH, NCH2, CH)
  z128 = jnp.zeros((NPAD, 128), jnp.float32)
  pad = jnp.zeros((128, 64), jnp.float32)
  W3p = jnp.concatenate([W3, pad], axis=1)
  S3p = jnp.concatenate([S3, pad], axis=1)
  b3p = jnp.concatenate([b3, jnp.zeros((64,), jnp.float32)])

  sup, slf = _make_mm_first(128, 128)(fea, W0, S0, b0.reshape(1, -1))
  agg = _make_spmm(128)(sup, src3, dst3, z128)
  sup, slf = _make_mm_mid(128, 128)(agg, slf, W1, S1, b1.reshape(1, -1))
  agg = _make_spmm(128)(sup, src3, dst3, z128)
  sup, slf = _make_mm_mid(128, 128)(agg, slf, W2, S2, b2.reshape(1, -1))
  agg = _make_spmm(128)(sup, src3, dst3, z128)
  sup, slf = _make_mm_mid(128, 128)(agg, slf, W3p, S3p, b3p.reshape(1, -1))
  agg = _make_spmm(128)(sup, src3, dst3, z128)
  return _make_final()(agg, slf)


# speedup vs baseline: 10.9959x; 1.0227x over previous
import functools

import jax
import jax.numpy as jnp
from jax import lax
from jax.experimental import pallas as pl
from jax.experimental.pallas import tpu as pltpu
from jax.experimental.pallas import tpu_sc as plsc

N = 10000
E = 320000
NC = 2
NS = 16
NW = NC * NS
EPT = E // NW
CH = 80
NCH = EPT // CH
NH = 5
NCH2 = NCH // NH
NPAD = 10112
RPT = NPAD // NS


@functools.lru_cache(maxsize=None)
def _make_spmm(D):
  mesh = plsc.VectorSubcoreMesh(core_axis_name="c", subcore_axis_name="s")

  @functools.partial(
      pl.kernel,
      out_type=jax.ShapeDtypeStruct((NC, NPAD, D), jnp.float32),
      mesh=mesh,
      scratch_types=[
          pltpu.VMEM((2, NCH2, CH), jnp.int32),
          pltpu.VMEM((2, NCH2, CH), jnp.int32),
          pltpu.VMEM((3, CH, D), jnp.float32),
          pltpu.VMEM_SHARED((NPAD, D), jnp.float32),
          [pltpu.SemaphoreType.DMA] * 3,
          [pltpu.SemaphoreType.DMA] * 3,
          pltpu.SemaphoreType.DMA,
      ],
  )
  def spmm(support, src5, dst5, zeros, out,
           src_v, dst_v, rows, acc, gsem, ssem, isem):
    cid = lax.axis_index("c")
    sid = lax.axis_index("s")
    wid = sid * NC + cid

    def g_issue(p, j, b):
      pltpu.async_copy(support.at[src_v.at[p, j]], rows.at[b], gsem[b])

    def g_wait(b):
      pltpu.make_async_copy(support.at[src_v.at[0, 0]], rows.at[b],
                            gsem[b]).wait()

    def s_issue(p, j, b):
      pltpu.async_copy(rows.at[b], acc.at[dst_v.at[p, j]], ssem[b], add=True)

    def s_wait(b):
      pltpu.make_async_copy(rows.at[b], acc.at[dst_v.at[0, 0]],
                            ssem[b]).wait()

    pltpu.sync_copy(src5.at[wid, 0], src_v.at[0])
    pltpu.sync_copy(dst5.at[wid, 0], dst_v.at[0])
    g_issue(0, 0, 0)
    g_issue(0, 1, 1)
    pltpu.sync_copy(zeros.at[pl.ds(sid * RPT, RPT)],
                    acc.at[pl.ds(sid * RPT, RPT)])
    plsc.subcore_barrier()

    for h in range(NH):
      p = h % 2
      if h + 1 < NH:
        pltpu.async_copy(src5.at[wid, h + 1], src_v.at[1 - p], isem)
        pltpu.async_copy(dst5.at[wid, h + 1], dst_v.at[1 - p], isem)
      g_wait(0)
      s_issue(p, 0, 0)
      g_issue(p, 2, 2)

      def body(i, carry):
        for k in range(3):
          j = 1 + 3 * i + k
          b = (1 + k) % 3
          bn = (b + 2) % 3
          g_wait(b)
          s_issue(p, j, b)
          s_wait(bn)
          g_issue(p, lax.rem(j + 2, NCH2), bn)
        return carry

      lax.fori_loop(0, (NCH2 - 1) // 3, body, 0)
      g_wait(1)
      g_wait(2)
      s_wait(0)
      if h + 1 < NH:
        pltpu.make_async_copy(src5.at[wid, 0], src_v.at[0], isem).wait()
        pltpu.make_async_copy(dst5.at[wid, 0], dst_v.at[0], isem).wait()
        g_issue(1 - p, 0, 0)
        g_issue(1 - p, 1, 1)
    plsc.subcore_barrier()
    pltpu.sync_copy(acc.at[pl.ds(sid * RPT, RPT)],
                    out.at[cid, pl.ds(sid * RPT, RPT)])

  return spmm


BN = 1000


def _self_body(x_ref, s_ref, b_ref, slf_ref):
  slf_ref[...] = (
      jnp.dot(x_ref[...], s_ref[...], preferred_element_type=jnp.float32)
      + b_ref[...])


def _comb_body(agg_ref, slf_ref, w_ref, o_ref):
  a = agg_ref[0] + agg_ref[1]
  o_ref[...] = (
      jnp.dot(a, w_ref[...], preferred_element_type=jnp.float32)
      + slf_ref[...])


def _comb_final_body(agg_ref, slf_ref, w_ref, o_ref):
  a = agg_ref[0] + agg_ref[1]
  x = (jnp.dot(a, w_ref[...], preferred_element_type=jnp.float32)
       + slf_ref[...])
  m = jnp.max(x, axis=1, keepdims=True)
  e = jnp.exp(x - m)
  lse = jnp.log(jnp.sum(e, axis=1, keepdims=True)) + m
  o_ref[...] = x - lse


@functools.lru_cache(maxsize=None)
def _make_self(DI, DO):
  return pl.pallas_call(
      _self_body,
      grid=(N // BN,),
      in_specs=[
          pl.BlockSpec((BN, DI), lambda i: (i, 0)),
          pl.BlockSpec((DI, DO), lambda i: (0, 0)),
          pl.BlockSpec((1, DO), lambda i: (0, 0)),
      ],
      out_specs=pl.BlockSpec((BN, DO), lambda i: (i, 0)),
      out_shape=jax.ShapeDtypeStruct((N, DO), jnp.float32),
  )


@functools.lru_cache(maxsize=None)
def _make_comb(DO, final):
  return pl.pallas_call(
      _comb_final_body if final else _comb_body,
      grid=(N // BN,),
      in_specs=[
          pl.BlockSpec((NC, BN, 128), lambda i: (0, i, 0)),
          pl.BlockSpec((BN, DO), lambda i: (i, 0)),
          pl.BlockSpec((128, DO), lambda i: (0, 0)),
      ],
      out_specs=pl.BlockSpec((BN, DO), lambda i: (i, 0)),
      out_shape=jax.ShapeDtypeStruct((N, DO), jnp.float32),
  )


def kernel(fea, adj, W0, S0, b0, W1, S1, b1, W2, S2, b2, W3, S3, b3):
  src3 = adj[0].reshape(NW, NH, NCH2, CH)
  dst3 = adj[1].reshape(NW, NH, NCH2, CH)
  z128 = jnp.zeros((NPAD, 128), jnp.float32)
  spmm = _make_spmm(128)

  x = fea
  for W, S, b, DO in ((W0, S0, b0, 128), (W1, S1, b1, 128),
                      (W2, S2, b2, 128)):
    slf = _make_self(128, DO)(x, S, b.reshape(1, -1))
    agg = spmm(x, src3, dst3, z128)
    x = _make_comb(DO, False)(agg, slf, W)
  slf = _make_self(128, 64)(x, S3, b3.reshape(1, -1))
  agg = spmm(x, src3, dst3, z128)
  return _make_comb(64, True)(agg, slf, W3)
